# baseline (speedup 1.0000x reference)
import jax
import jax.numpy as jnp
from jax.experimental import pallas as pl
from jax.experimental.pallas import tpu as pltpu

_KH = _KW = 3
_PAD = 1
_BN_EPS = 1e-5


def _conv_stats_kernel(x_ref, w_ref, y_ref, stats_ref, xf_ref, col_ref):
    IM, HW, Cin = x_ref.shape
    _, Hs, Wp, _ = xf_ref.shape
    H = Hs - 3
    W = HW // H
    HWp = H * Wp
    Cout = stats_ref.shape[2]

    for j in range(IM):
        xf_ref[j, 0:1, :, :] = jnp.zeros((1, Wp, Cin), jnp.float32)
        xf_ref[j, H + 1:H + 3, :, :] = jnp.zeros((2, Wp, Cin), jnp.float32)
        xf_ref[j, :, 0:1, :] = jnp.zeros((Hs, 1, Cin), jnp.float32)
        xf_ref[j, :, W + 1:Wp, :] = jnp.zeros((Hs, Wp - W - 1, Cin),
                                              jnp.float32)
        xf_ref[j, 1:H + 1, 1:W + 1, :] = x_ref[j].reshape(H, W, Cin)

        for dy in range(_KH):
            col_ref[j, :, :, dy * Cin:(dy + 1) * Cin] = (
                xf_ref[j, dy:dy + H + 1, :, :])

        z = jnp.dot(col_ref[j].reshape((H + 1) * Wp, _KH * Cin), w_ref[...],
                    preferred_element_type=jnp.float32)

        y = (z[0:HWp, 0:Cout]
             + z[1:HWp + 1, Cout:2 * Cout]
             + z[2:HWp + 2, 2 * Cout:3 * Cout])

        y_ref[j * HWp:(j + 1) * HWp, :] = y.astype(jnp.bfloat16)
        y2 = y * y
        g = y.reshape(H, Wp, Cout)[:, W:Wp, :].reshape(H * (Wp - W), Cout)
        g2 = y2.reshape(H, Wp, Cout)[:, W:Wp, :].reshape(H * (Wp - W), Cout)
        s1 = (jnp.sum(y, axis=0, keepdims=True)
              - jnp.sum(g, axis=0, keepdims=True))
        s2 = (jnp.sum(y2, axis=0, keepdims=True)
              - jnp.sum(g2, axis=0, keepdims=True))
        stats_ref[j] = jnp.concatenate([s1, s2], axis=0)


def _bn_relu_kernel(y_ref, ab_ref, o_ref):
    _, H, Wp, Cout = y_ref.shape
    W = o_ref.shape[2]
    scale = ab_ref[0:1, :]
    shift = ab_ref[1:2, :]
    y = y_ref[0].reshape(H * Wp, Cout).astype(jnp.float32)
    o = jnp.maximum(y * scale + shift, 0.0)
    o_ref[...] = o.reshape(H, Wp, Cout)[:, :W, :].reshape(1, H, W, Cout)


def kernel(x_nchw, weight, bias):
    del bias
    N, Cin, H, W = x_nchw.shape
    Cout = weight.shape[0]
    HW = H * W
    Wp = 1
    while Wp < W + 2 * _PAD:
        Wp *= 2
    HWp = H * Wp
    rows = N * HW
    rows_p = N * HWp
    IM = 2 if N % 2 == 0 else 1

    x_rows = jnp.transpose(x_nchw.reshape(N, Cin, HW), (0, 2, 1))
    w3 = jnp.transpose(weight, (2, 1, 3, 0)).reshape(_KH * Cin, _KW * Cout)

    y_conv, part_stats = pl.pallas_call(
        _conv_stats_kernel,
        out_shape=(jax.ShapeDtypeStruct((rows_p, Cout), jnp.bfloat16),
                   jax.ShapeDtypeStruct((N, 2, Cout), jnp.float32)),
        grid=(N // IM,),
        in_specs=[
            pl.BlockSpec((IM, HW, Cin), lambda i: (i, 0, 0)),
            pl.BlockSpec((_KH * Cin, _KW * Cout), lambda i: (0, 0)),
        ],
        out_specs=[
            pl.BlockSpec((IM * HWp, Cout), lambda i: (i, 0)),
            pl.BlockSpec((IM, 2, Cout), lambda i: (i, 0, 0)),
        ],
        scratch_shapes=[
            pltpu.VMEM((IM, H + 3, Wp, Cin), jnp.float32),
            pltpu.VMEM((IM, H + 1, Wp, _KH * Cin), jnp.float32),
        ],
        compiler_params=pltpu.CompilerParams(
            dimension_semantics=("parallel",)),
    )(x_rows, w3)

    sums = jnp.sum(part_stats, axis=0)
    mean = sums[0] / rows
    var = sums[1] / rows - mean * mean
    inv_std = jax.lax.rsqrt(var + _BN_EPS)
    ab = jnp.stack([inv_std, -mean * inv_std], axis=0)

    y4 = y_conv.reshape(N, H, Wp, Cout)
    out_nhwc = pl.pallas_call(
        _bn_relu_kernel,
        out_shape=jax.ShapeDtypeStruct((N, H, W, Cout), jnp.float32),
        grid=(N,),
        in_specs=[
            pl.BlockSpec((1, H, Wp, Cout), lambda i: (i, 0, 0, 0)),
            pl.BlockSpec((2, Cout), lambda i: (0, 0)),
        ],
        out_specs=pl.BlockSpec((1, H, W, Cout), lambda i: (i, 0, 0, 0)),
        compiler_params=pltpu.CompilerParams(
            dimension_semantics=("parallel",)),
    )(y4, ab)

    return jnp.transpose(out_nhwc, (0, 3, 1, 2))


# speedup vs baseline: 1.1061x; 1.1061x over previous
import jax
import jax.numpy as jnp
from jax.experimental import pallas as pl
from jax.experimental.pallas import tpu as pltpu

_KH = _KW = 3
_PAD = 1
_BN_EPS = 1e-5


def _conv_stats_kernel(x_ref, w_ref, y_ref, stats_ref, xf_ref, col_ref):
    IM, HW, Cin = x_ref.shape
    _, Hs, Wp, _ = xf_ref.shape
    H = Hs - 3
    W = HW // H
    HWp = H * Wp
    Cout = stats_ref.shape[2]

    for j in range(IM):
        xf_ref[j, 0:1, :, :] = jnp.zeros((1, Wp, Cin), jnp.float32)
        xf_ref[j, H + 1:H + 3, :, :] = jnp.zeros((2, Wp, Cin), jnp.float32)
        xf_ref[j, :, 0:1, :] = jnp.zeros((Hs, 1, Cin), jnp.float32)
        xf_ref[j, :, W + 1:Wp, :] = jnp.zeros((Hs, Wp - W - 1, Cin),
                                              jnp.float32)
        xf_ref[j, 1:H + 1, 1:W + 1, :] = x_ref[j].reshape(H, W, Cin)

        for dy in range(_KH):
            col_ref[j, :, :, dy * Cin:(dy + 1) * Cin] = (
                xf_ref[j, dy:dy + H + 1, :, :])

        z = jnp.dot(col_ref[j].reshape((H + 1) * Wp, _KH * Cin), w_ref[...],
                    preferred_element_type=jnp.float32)

        y = (z[0:HWp, 0:Cout]
             + z[1:HWp + 1, Cout:2 * Cout]
             + z[2:HWp + 2, 2 * Cout:3 * Cout])

        y_ref[j * HWp:(j + 1) * HWp, :] = y.astype(jnp.bfloat16)
        y2 = y * y
        g = y.reshape(H, Wp, Cout)[:, W:Wp, :].reshape(H * (Wp - W), Cout)
        g2 = y2.reshape(H, Wp, Cout)[:, W:Wp, :].reshape(H * (Wp - W), Cout)
        s1 = (jnp.sum(y, axis=0, keepdims=True)
              - jnp.sum(g, axis=0, keepdims=True))
        s2 = (jnp.sum(y2, axis=0, keepdims=True)
              - jnp.sum(g2, axis=0, keepdims=True))
        stats_ref[j] = jnp.concatenate([s1, s2], axis=0)


def _bn_relu_kernel(y_ref, ab_ref, o_ref):
    IM, H, Wp, Cout = y_ref.shape
    W = o_ref.shape[2]
    scale = ab_ref[0:1, :]
    shift = ab_ref[1:2, :]
    y = y_ref[...].reshape(IM * H * Wp, Cout).astype(jnp.float32)
    o = jnp.maximum(y * scale + shift, 0.0)
    o_ref[...] = o.reshape(IM, H, Wp, Cout)[:, :, :W, :]


def kernel(x_nchw, weight, bias):
    del bias
    N, Cin, H, W = x_nchw.shape
    Cout = weight.shape[0]
    HW = H * W
    Wp = 1
    while Wp < W + 2 * _PAD:
        Wp *= 2
    HWp = H * Wp
    rows = N * HW
    rows_p = N * HWp
    IM = 2 if N % 2 == 0 else 1

    x_rows = jnp.transpose(x_nchw.reshape(N, Cin, HW), (0, 2, 1))
    w3 = jnp.transpose(weight, (2, 1, 3, 0)).reshape(_KH * Cin, _KW * Cout)

    y_conv, part_stats = pl.pallas_call(
        _conv_stats_kernel,
        out_shape=(jax.ShapeDtypeStruct((rows_p, Cout), jnp.bfloat16),
                   jax.ShapeDtypeStruct((N, 2, Cout), jnp.float32)),
        grid=(N // IM,),
        in_specs=[
            pl.BlockSpec((IM, HW, Cin), lambda i: (i, 0, 0)),
            pl.BlockSpec((_KH * Cin, _KW * Cout), lambda i: (0, 0)),
        ],
        out_specs=[
            pl.BlockSpec((IM * HWp, Cout), lambda i: (i, 0)),
            pl.BlockSpec((IM, 2, Cout), lambda i: (i, 0, 0)),
        ],
        scratch_shapes=[
            pltpu.VMEM((IM, H + 3, Wp, Cin), jnp.float32),
            pltpu.VMEM((IM, H + 1, Wp, _KH * Cin), jnp.float32),
        ],
        compiler_params=pltpu.CompilerParams(
            dimension_semantics=("parallel",)),
    )(x_rows, w3)

    sums = jnp.sum(part_stats, axis=0)
    mean = sums[0] / rows
    var = sums[1] / rows - mean * mean
    inv_std = jax.lax.rsqrt(var + _BN_EPS)
    ab = jnp.stack([inv_std, -mean * inv_std], axis=0)

    y4 = y_conv.reshape(N, H, Wp, Cout)
    out_nhwc = pl.pallas_call(
        _bn_relu_kernel,
        out_shape=jax.ShapeDtypeStruct((N, H, W, Cout), jnp.float32),
        grid=(N // IM,),
        in_specs=[
            pl.BlockSpec((IM, H, Wp, Cout), lambda i: (i, 0, 0, 0)),
            pl.BlockSpec((2, Cout), lambda i: (0, 0)),
        ],
        out_specs=pl.BlockSpec((IM, H, W, Cout), lambda i: (i, 0, 0, 0)),
        compiler_params=pltpu.CompilerParams(
            dimension_semantics=("parallel",)),
    )(y4, ab)

    return jnp.transpose(out_nhwc, (0, 3, 1, 2))


# speedup vs baseline: 1.1285x; 1.0203x over previous
import jax
import jax.numpy as jnp
from jax.experimental import pallas as pl
from jax.experimental.pallas import tpu as pltpu

_KH = _KW = 3
_PAD = 1
_BN_EPS = 1e-5


def _conv_stats_kernel(x_ref, w_ref, y_ref, stats_ref, xf_ref, col_ref):
    IM, HW, Cin = x_ref.shape
    _, Hs, Wp, _ = xf_ref.shape
    H = Hs - 3
    W = HW // H
    HWp = H * Wp
    Cout = stats_ref.shape[2]

    for j in range(IM):
        xf_ref[j, 0:1, :, :] = jnp.zeros((1, Wp, Cin), jnp.float32)
        xf_ref[j, H + 1:H + 3, :, :] = jnp.zeros((2, Wp, Cin), jnp.float32)
        xf_ref[j, :, 0:1, :] = jnp.zeros((Hs, 1, Cin), jnp.float32)
        xf_ref[j, :, W + 1:Wp, :] = jnp.zeros((Hs, Wp - W - 1, Cin),
                                              jnp.float32)
        xf_ref[j, 1:H + 1, 1:W + 1, :] = x_ref[j].reshape(H, W, Cin)

        for dy in range(_KH):
            col_ref[j, :, :, dy * Cin:(dy + 1) * Cin] = (
                xf_ref[j, dy:dy + H + 1, :, :])

        z = jnp.dot(col_ref[j].reshape((H + 1) * Wp, _KH * Cin), w_ref[...],
                    preferred_element_type=jnp.float32)

        y = (z[0:HWp, 0:Cout]
             + z[1:HWp + 1, Cout:2 * Cout]
             + z[2:HWp + 2, 2 * Cout:3 * Cout])

        y_ref[j * HWp:(j + 1) * HWp, :] = y.astype(jnp.bfloat16)
        y2 = y * y
        g = y.reshape(H, Wp, Cout)[:, W:Wp, :].reshape(H * (Wp - W), Cout)
        g2 = y2.reshape(H, Wp, Cout)[:, W:Wp, :].reshape(H * (Wp - W), Cout)
        s1 = (jnp.sum(y, axis=0, keepdims=True)
              - jnp.sum(g, axis=0, keepdims=True))
        s2 = (jnp.sum(y2, axis=0, keepdims=True)
              - jnp.sum(g2, axis=0, keepdims=True))
        stats_ref[j] = jnp.concatenate([s1, s2], axis=0)


def _bn_relu_kernel(y_ref, ab_ref, o_ref):
    IM, H, Wp, Cout = y_ref.shape
    W = o_ref.shape[2]
    scale = ab_ref[0:1, :]
    shift = ab_ref[1:2, :]
    y = y_ref[...].reshape(IM * H * Wp, Cout).astype(jnp.float32)
    o = jnp.maximum(y * scale + shift, 0.0)
    o_ref[...] = o.reshape(IM, H, Wp, Cout)[:, :, :W, :]


def kernel(x_nchw, weight, bias):
    del bias
    N, Cin, H, W = x_nchw.shape
    Cout = weight.shape[0]
    HW = H * W
    Wp = 1
    while Wp < W + 2 * _PAD:
        Wp *= 2
    HWp = H * Wp
    rows = N * HW
    rows_p = N * HWp
    IM = 2 if N % 2 == 0 else 1

    x_rows = jnp.transpose(x_nchw.reshape(N, Cin, HW), (0, 2, 1))
    w3 = jnp.transpose(weight, (2, 1, 3, 0)).reshape(_KH * Cin, _KW * Cout)

    y_conv, part_stats = pl.pallas_call(
        _conv_stats_kernel,
        out_shape=(jax.ShapeDtypeStruct((rows_p, Cout), jnp.bfloat16),
                   jax.ShapeDtypeStruct((N, 2, Cout), jnp.float32)),
        grid=(N // IM,),
        in_specs=[
            pl.BlockSpec((IM, HW, Cin), lambda i: (i, 0, 0)),
            pl.BlockSpec((_KH * Cin, _KW * Cout), lambda i: (0, 0)),
        ],
        out_specs=[
            pl.BlockSpec((IM * HWp, Cout), lambda i: (i, 0)),
            pl.BlockSpec((IM, 2, Cout), lambda i: (i, 0, 0)),
        ],
        scratch_shapes=[
            pltpu.VMEM((IM, H + 3, Wp, Cin), jnp.float32),
            pltpu.VMEM((IM, H + 1, Wp, _KH * Cin), jnp.float32),
        ],
        compiler_params=pltpu.CompilerParams(
            dimension_semantics=("parallel",)),
    )(x_rows, w3)

    sums = jnp.sum(part_stats, axis=0)
    mean = sums[0] / rows
    var = sums[1] / rows - mean * mean
    inv_std = jax.lax.rsqrt(var + _BN_EPS)
    ab = jnp.stack([inv_std, -mean * inv_std], axis=0)

    IM2 = 4 if N % 4 == 0 else IM
    y4 = y_conv.reshape(N, H, Wp, Cout)
    out_nhwc = pl.pallas_call(
        _bn_relu_kernel,
        out_shape=jax.ShapeDtypeStruct((N, H, W, Cout), jnp.float32),
        grid=(N // IM2,),
        in_specs=[
            pl.BlockSpec((IM2, H, Wp, Cout), lambda i: (i, 0, 0, 0)),
            pl.BlockSpec((2, Cout), lambda i: (0, 0)),
        ],
        out_specs=pl.BlockSpec((IM2, H, W, Cout), lambda i: (i, 0, 0, 0)),
        compiler_params=pltpu.CompilerParams(
            dimension_semantics=("parallel",)),
    )(y4, ab)

    return jnp.transpose(out_nhwc, (0, 3, 1, 2))


# speedup vs baseline: 1.1704x; 1.0371x over previous
import jax
import jax.numpy as jnp
from jax.experimental import pallas as pl
from jax.experimental.pallas import tpu as pltpu

_KH = _KW = 3
_PAD = 1
_BN_EPS = 1e-5


def _conv_stats_kernel(x_ref, w_ref, y_ref, stats_ref, xf_ref, col_ref):
    IM, HW, Cin = x_ref.shape
    _, Hs, Wp, _ = xf_ref.shape
    H = Hs - 3
    W = HW // H
    HWp = H * Wp
    Cout = stats_ref.shape[2]

    for j in range(IM):
        xf_ref[j, 0:1, :, :] = jnp.zeros((1, Wp, Cin), jnp.float32)
        xf_ref[j, H + 1:H + 3, :, :] = jnp.zeros((2, Wp, Cin), jnp.float32)
        xf_ref[j, :, 0:1, :] = jnp.zeros((Hs, 1, Cin), jnp.float32)
        xf_ref[j, :, W + 1:Wp, :] = jnp.zeros((Hs, Wp - W - 1, Cin),
                                              jnp.float32)
        xf_ref[j, 1:H + 1, 1:W + 1, :] = x_ref[j].reshape(H, W, Cin)

        for dy in range(_KH):
            col_ref[j, :, :, dy * Cin:(dy + 1) * Cin] = (
                xf_ref[j, dy:dy + H + 1, :, :])

        z = jnp.dot(col_ref[j].reshape((H + 1) * Wp, _KH * Cin), w_ref[...],
                    preferred_element_type=jnp.float32)

        y = (z[0:HWp, 0:Cout]
             + z[1:HWp + 1, Cout:2 * Cout]
             + z[2:HWp + 2, 2 * Cout:3 * Cout])

        y_ref[j * HWp:(j + 1) * HWp, :] = y.astype(jnp.bfloat16)
        y2 = y * y
        g = y.reshape(H, Wp, Cout)[:, W:Wp, :].reshape(H * (Wp - W), Cout)
        g2 = y2.reshape(H, Wp, Cout)[:, W:Wp, :].reshape(H * (Wp - W), Cout)
        s1 = (jnp.sum(y, axis=0, keepdims=True)
              - jnp.sum(g, axis=0, keepdims=True))
        s2 = (jnp.sum(y2, axis=0, keepdims=True)
              - jnp.sum(g2, axis=0, keepdims=True))
        stats_ref[j] = jnp.concatenate([s1, s2], axis=0)


def _bn_relu_kernel(y_ref, ab_ref, o_ref):
    IM, H, Wp, Cout = y_ref.shape
    W = o_ref.shape[2]
    scale = ab_ref[0:1, :]
    shift = ab_ref[1:2, :]
    y = y_ref[...].reshape(IM * H * Wp, Cout).astype(jnp.float32)
    o = jnp.maximum(y * scale + shift, 0.0)
    o_ref[...] = o.reshape(IM, H, Wp, Cout)[:, :, :W, :]


def kernel(x_nchw, weight, bias):
    del bias
    N, Cin, H, W = x_nchw.shape
    Cout = weight.shape[0]
    HW = H * W
    Wp = 1
    while Wp < W + 2 * _PAD:
        Wp *= 2
    HWp = H * Wp
    rows = N * HW
    rows_p = N * HWp
    IM = 2 if N % 2 == 0 else 1

    x_rows = jnp.transpose(x_nchw.reshape(N, Cin, HW), (0, 2, 1))
    w3 = jnp.transpose(weight, (2, 1, 3, 0)).reshape(_KH * Cin, _KW * Cout)

    y_conv, part_stats = pl.pallas_call(
        _conv_stats_kernel,
        out_shape=(jax.ShapeDtypeStruct((rows_p, Cout), jnp.bfloat16),
                   jax.ShapeDtypeStruct((N, 2, Cout), jnp.float32)),
        grid=(N // IM,),
        in_specs=[
            pl.BlockSpec((IM, HW, Cin), lambda i: (i, 0, 0)),
            pl.BlockSpec((_KH * Cin, _KW * Cout), lambda i: (0, 0)),
        ],
        out_specs=[
            pl.BlockSpec((IM * HWp, Cout), lambda i: (i, 0)),
            pl.BlockSpec((IM, 2, Cout), lambda i: (i, 0, 0)),
        ],
        scratch_shapes=[
            pltpu.VMEM((IM, H + 3, Wp, Cin), jnp.float32),
            pltpu.VMEM((IM, H + 1, Wp, _KH * Cin), jnp.float32),
        ],
        compiler_params=pltpu.CompilerParams(
            dimension_semantics=("parallel",)),
    )(x_rows, w3)

    sums = jnp.sum(part_stats, axis=0)
    mean = sums[0] / rows
    var = sums[1] / rows - mean * mean
    inv_std = jax.lax.rsqrt(var + _BN_EPS)
    ab = jnp.stack([inv_std, -mean * inv_std], axis=0)

    IM2 = 8 if N % 8 == 0 else (4 if N % 4 == 0 else IM)
    y4 = y_conv.reshape(N, H, Wp, Cout)
    out_nhwc = pl.pallas_call(
        _bn_relu_kernel,
        out_shape=jax.ShapeDtypeStruct((N, H, W, Cout), jnp.float32),
        grid=(N // IM2,),
        in_specs=[
            pl.BlockSpec((IM2, H, Wp, Cout), lambda i: (i, 0, 0, 0)),
            pl.BlockSpec((2, Cout), lambda i: (0, 0)),
        ],
        out_specs=pl.BlockSpec((IM2, H, W, Cout), lambda i: (i, 0, 0, 0)),
        compiler_params=pltpu.CompilerParams(
            dimension_semantics=("parallel",)),
    )(y4, ab)

    return jnp.transpose(out_nhwc, (0, 3, 1, 2))
